# baseline (speedup 1.0000x reference)
import functools

import jax
import jax.numpy as jnp
from jax import lax
from jax.experimental import pallas as pl
from jax.experimental.pallas import tpu as pltpu
from jax.experimental.pallas import tpu_sc as plsc

N_NODES = 10000
N_EDGES = 320000
D_IN = 256
D_HID = 128

NC = 2
NS = 16
NW = NC * NS

NPAD = 10240
EPAD = 327680
EPT = EPAD // NW
GPT = EPT // 128
ROWS_PT = NPAD // NS

MB = 2048
GRID_M = NPAD // MB

_mesh = plsc.VectorSubcoreMesh(
    core_axis_name="c", subcore_axis_name="s", num_cores=NC, num_subcores=NS)


@functools.partial(
    pl.kernel,
    out_type=jax.ShapeDtypeStruct((NW, NPAD // 16, 16), jnp.float32),
    mesh=_mesh,
    compiler_params=pltpu.CompilerParams(needs_layout_passes=False),
    scratch_types=[
        pltpu.VMEM((NPAD // 16, 16), jnp.float32),
        pltpu.VMEM((2048,), jnp.int32),
    ],
)
def _deg_kernel(dst_hbm, out_hbm, acc, idxb):
    cid = lax.axis_index("c")
    sid = lax.axis_index("s")
    wid = cid * NS + sid

    z16 = jnp.zeros((16,), jnp.float32)

    def zero_body(i, _):
        acc[i, pl.ds(0, 16)] = z16
        return 0

    lax.fori_loop(0, NPAD // 16, zero_body, 0)

    ones16 = jnp.ones((16,), jnp.float32)

    def chunk_body(c, _):
        pltpu.sync_copy(dst_hbm.at[pl.ds(wid * EPT + c * 2048, 2048)], idxb)

        def grp_body(i, _):
            idx = idxb[pl.ds(i * 16, 16)]
            plsc.addupdate_scatter(
                acc, [lax.shift_right_logical(idx, 4),
                      jnp.bitwise_and(idx, 15)], ones16)
            return 0

        lax.fori_loop(0, 128, grp_body, 0)
        return 0

    lax.fori_loop(0, EPT // 2048, chunk_body, 0)
    pltpu.sync_copy(acc, out_hbm.at[wid])


G_SC0 = 112
G_SC1 = 2 * GPT - G_SC0


@functools.partial(
    pl.kernel,
    out_type=jax.ShapeDtypeStruct((NC, NPAD, D_HID), jnp.float32),
    mesh=_mesh,
    compiler_params=pltpu.CompilerParams(needs_layout_passes=False),
    scratch_types=[
        pltpu.VMEM((16, 128), jnp.int32),
        pltpu.VMEM((16, 128), jnp.int32),
        pltpu.VMEM((256, D_HID), jnp.float32),
        pltpu.VMEM_SHARED((NPAD, D_HID), jnp.float32),
        pltpu.SemaphoreType.DMA,
        pltpu.SemaphoreType.DMA,
        pltpu.SemaphoreType.DMA,
    ],
)
def _scatter_kernel(hs_hbm, src_hbm, dst_hbm, out_hbm,
                    sidx, didx, bufs, acc, gsem, ssem, isem):
    cid = lax.axis_index("c")
    sid = lax.axis_index("s")
    gcnt = jnp.where(cid == 0, G_SC0, G_SC1)
    gbase = jnp.where(cid == 0, sid * G_SC0, NS * G_SC0 + sid * G_SC1)
    nblk = gcnt // 8

    z16 = jnp.zeros((16,), jnp.float32)

    def zero_body(i, _):
        bufs[i >> 3, pl.ds((i & 7) * 16, 16)] = z16
        return 0

    lax.fori_loop(0, 128 * (D_HID // 16), zero_body, 0)

    def zcopy_body(z, _):
        pltpu.sync_copy(bufs.at[pl.ds(0, 128)],
                        acc.at[pl.ds(sid * ROWS_PT + z * 128, 128)])
        return 0

    lax.fori_loop(0, ROWS_PT // 128, zcopy_body, 0)
    plsc.subcore_barrier()

    pltpu.sync_copy(src_hbm.at[pl.ds(gbase, 8)], sidx.at[pl.ds(0, 8)])
    pltpu.sync_copy(dst_hbm.at[pl.ds(gbase, 8)], didx.at[pl.ds(0, 8)])
    pltpu.async_copy(hs_hbm.at[sidx.at[0]], bufs.at[pl.ds(0, 128)], gsem)

    def body(j, _):
        u = (j % 2) * 128
        blk = j // 8
        r = j - blk * 8
        slot = (blk % 2) * 8

        pltpu.make_async_copy(hs_hbm.at[sidx.at[0]],
                              bufs.at[pl.ds(u, 128)], gsem).wait()

        @pl.when(j >= 1)
        def _():
            pltpu.make_async_copy(bufs.at[pl.ds(128 - u, 128)],
                                  acc.at[didx.at[0]], ssem).wait()

        @pl.when(jnp.logical_and(r == 0, blk + 1 < nblk))
        def _():
            pltpu.async_copy(src_hbm.at[pl.ds(gbase + (blk + 1) * 8, 8)],
                             sidx.at[pl.ds(8 - slot, 8)], isem)
            pltpu.async_copy(dst_hbm.at[pl.ds(gbase + (blk + 1) * 8, 8)],
                             didx.at[pl.ds(8 - slot, 8)], isem)

        @pl.when(jnp.logical_and(r == 7, j + 1 < gcnt))
        def _():
            pltpu.make_async_copy(src_hbm.at[pl.ds(gbase, 8)],
                                  sidx.at[pl.ds(0, 8)], isem).wait()
            pltpu.make_async_copy(dst_hbm.at[pl.ds(gbase, 8)],
                                  didx.at[pl.ds(0, 8)], isem).wait()

        @pl.when(j + 1 < gcnt)
        def _():
            blk1 = (j + 1) // 8
            srow = (blk1 % 2) * 8 + (j + 1 - blk1 * 8)
            pltpu.async_copy(hs_hbm.at[sidx.at[srow]],
                             bufs.at[pl.ds(128 - u, 128)], gsem)

        pltpu.async_copy(bufs.at[pl.ds(u, 128)],
                         acc.at[didx.at[slot + r]], ssem, add=True)
        return 0

    lax.fori_loop(0, gcnt, body, 0)
    pltpu.make_async_copy(bufs.at[pl.ds(128, 128)],
                          acc.at[didx.at[0]], ssem).wait()

    plsc.subcore_barrier()

    def wb_body(z, _):
        r0 = sid * ROWS_PT + z * 128
        pltpu.sync_copy(acc.at[pl.ds(r0, 128)], bufs.at[pl.ds(0, 128)])
        pltpu.sync_copy(bufs.at[pl.ds(0, 128)],
                        out_hbm.at[cid].at[pl.ds(r0, 128)])
        return 0

    lax.fori_loop(0, ROWS_PT // 128, wb_body, 0)


def _dis_dinv(degp):
    deg = jnp.sum(degp, axis=0) + 1.0
    dis = lax.rsqrt(deg)[:, None]
    dinv = (1.0 / deg)[:, None]
    return dis, dinv


def _mm1_body(x_ref, w_ref, degp_ref, h_ref, hs_ref):
    dis, _ = _dis_dinv(degp_ref[...])
    h = jnp.dot(x_ref[...], w_ref[...], preferred_element_type=jnp.float32)
    h_ref[...] = h
    hs_ref[...] = h * dis


def _mid_body(s_ref, h_ref, degp_ref, w_ref, b_ref, h2_ref, hs2_ref):
    dis, dinv = _dis_dinv(degp_ref[...])
    s = s_ref[0] + s_ref[1]
    a = jnp.maximum(s * dis + h_ref[...] * dinv + b_ref[...], 0.0)
    h2 = jnp.dot(a, w_ref[...], preferred_element_type=jnp.float32)
    h2_ref[...] = h2
    hs2_ref[...] = h2 * dis


def _out_body(s_ref, h_ref, degp_ref, b_ref, o_ref):
    dis, dinv = _dis_dinv(degp_ref[...])
    s = s_ref[0] + s_ref[1]
    o_ref[...] = jnp.maximum(s * dis + h_ref[...] * dinv + b_ref[...], 0.0)


def _row_spec(width):
    return pl.BlockSpec((MB, width), lambda i: (i, 0))


_degp_spec = pl.BlockSpec((NW, MB), lambda i: (0, i))
_s_spec = pl.BlockSpec((NC, MB, D_HID), lambda i: (0, i, 0))
_b_spec = pl.BlockSpec((1, D_HID), lambda i: (0, 0))


def _mm1(x, w1, degp):
    return pl.pallas_call(
        _mm1_body,
        grid=(GRID_M,),
        in_specs=[_row_spec(D_IN),
                  pl.BlockSpec((D_IN, D_HID), lambda i: (0, 0)),
                  _degp_spec],
        out_specs=[_row_spec(D_HID), _row_spec(D_HID)],
        out_shape=[jax.ShapeDtypeStruct((N_NODES, D_HID), jnp.float32),
                   jax.ShapeDtypeStruct((N_NODES, D_HID), jnp.float32)],
    )(x, w1, degp)


def _mid(s, h, degp, w2, b1):
    return pl.pallas_call(
        _mid_body,
        grid=(GRID_M,),
        in_specs=[_s_spec, _row_spec(D_HID), _degp_spec,
                  pl.BlockSpec((D_HID, D_HID), lambda i: (0, 0)), _b_spec],
        out_specs=[_row_spec(D_HID), _row_spec(D_HID)],
        out_shape=[jax.ShapeDtypeStruct((N_NODES, D_HID), jnp.float32),
                   jax.ShapeDtypeStruct((N_NODES, D_HID), jnp.float32)],
    )(s, h, degp, w2, b1)


def _out(s, h, degp, b2):
    return pl.pallas_call(
        _out_body,
        grid=(GRID_M,),
        in_specs=[_s_spec, _row_spec(D_HID), _degp_spec, _b_spec],
        out_specs=_row_spec(D_HID),
        out_shape=jax.ShapeDtypeStruct((N_NODES, D_HID), jnp.float32),
    )(s, h, degp, b2)


def kernel(x, edge_index, y, W1, b1, W2, b2):
    del y
    src = edge_index[0].astype(jnp.int32)
    dst = edge_index[1].astype(jnp.int32)
    pad_e = EPAD - N_EDGES
    src_p = jnp.concatenate([src, jnp.zeros((pad_e,), jnp.int32)])
    dst_p = jnp.concatenate([dst, jnp.full((pad_e,), NPAD - 1, jnp.int32)])
    src_m = src_p.reshape(EPAD // 128, 128)
    dst_m = dst_p.reshape(EPAD // 128, 128)

    b1r = b1.reshape(1, D_HID)
    b2r = b2.reshape(1, D_HID)

    degp = _deg_kernel(dst_p).reshape(NW, NPAD)
    h1, hs1 = _mm1(x, W1, degp)
    s1 = _scatter_kernel(hs1, src_m, dst_m)
    h2, hs2 = _mid(s1, h1, degp, W2, b1r)
    s2 = _scatter_kernel(hs2, src_m, dst_m)
    return _out(s2, h2, degp, b2r)


# speedup vs baseline: 1.1927x; 1.1927x over previous
import functools

import jax
import jax.numpy as jnp
from jax import lax
from jax.experimental import pallas as pl
from jax.experimental.pallas import tpu as pltpu
from jax.experimental.pallas import tpu_sc as plsc

N_NODES = 10000
N_EDGES = 320000
D_IN = 256
D_HID = 128

NC = 2
NS = 16
NW = NC * NS

NPAD = 10240
EPAD = 327680
EPT = EPAD // NW
GPT = EPT // 128
ROWS_PT = NPAD // NS

MB = 2048
GRID_M = NPAD // MB

_mesh = plsc.VectorSubcoreMesh(
    core_axis_name="c", subcore_axis_name="s", num_cores=NC, num_subcores=NS)


@functools.partial(
    pl.kernel,
    out_type=jax.ShapeDtypeStruct((NW, NPAD // 16, 16), jnp.float32),
    mesh=_mesh,
    compiler_params=pltpu.CompilerParams(needs_layout_passes=False),
    scratch_types=[
        pltpu.VMEM((NPAD // 16, 16), jnp.float32),
        pltpu.VMEM((2048,), jnp.int32),
    ],
)
def _deg_kernel(dst_hbm, out_hbm, acc, idxb):
    cid = lax.axis_index("c")
    sid = lax.axis_index("s")
    wid = cid * NS + sid

    z16 = jnp.zeros((16,), jnp.float32)

    def zero_body(i, _):
        acc[i, pl.ds(0, 16)] = z16
        return 0

    lax.fori_loop(0, NPAD // 16, zero_body, 0)

    ones16 = jnp.ones((16,), jnp.float32)

    def chunk_body(c, _):
        pltpu.sync_copy(dst_hbm.at[pl.ds(wid * EPT + c * 2048, 2048)], idxb)

        def grp_body(i, _):
            idx = idxb[pl.ds(i * 16, 16)]
            plsc.addupdate_scatter(
                acc, [lax.shift_right_logical(idx, 4),
                      jnp.bitwise_and(idx, 15)], ones16)
            return 0

        lax.fori_loop(0, 128, grp_body, 0)
        return 0

    lax.fori_loop(0, EPT // 2048, chunk_body, 0)
    pltpu.sync_copy(acc, out_hbm.at[wid])


G_SC0 = 144
G_SC1 = 2 * GPT - G_SC0


@functools.partial(
    pl.kernel,
    out_type=jax.ShapeDtypeStruct((NC, NPAD, D_HID), jnp.float32),
    mesh=_mesh,
    compiler_params=pltpu.CompilerParams(needs_layout_passes=False),
    scratch_types=[
        pltpu.VMEM((16, 128), jnp.int32),
        pltpu.VMEM((16, 128), jnp.int32),
        pltpu.VMEM((256, D_HID), jnp.float32),
        pltpu.VMEM_SHARED((NPAD, D_HID), jnp.float32),
        pltpu.SemaphoreType.DMA,
        pltpu.SemaphoreType.DMA,
        pltpu.SemaphoreType.DMA,
    ],
)
def _scatter_kernel(hs_hbm, src_hbm, dst_hbm, out_hbm,
                    sidx, didx, bufs, acc, gsem, ssem, isem):
    cid = lax.axis_index("c")
    sid = lax.axis_index("s")
    gcnt = jnp.where(cid == 0, G_SC0, G_SC1)
    gbase = jnp.where(cid == 0, sid * G_SC0, NS * G_SC0 + sid * G_SC1)
    nblk = gcnt // 8

    z16 = jnp.zeros((16,), jnp.float32)

    def zero_body(i, _):
        bufs[i >> 3, pl.ds((i & 7) * 16, 16)] = z16
        return 0

    lax.fori_loop(0, 128 * (D_HID // 16), zero_body, 0)

    def zcopy_body(z, _):
        pltpu.sync_copy(bufs.at[pl.ds(0, 128)],
                        acc.at[pl.ds(sid * ROWS_PT + z * 128, 128)])
        return 0

    lax.fori_loop(0, ROWS_PT // 128, zcopy_body, 0)
    plsc.subcore_barrier()

    pltpu.sync_copy(src_hbm.at[pl.ds(gbase, 8)], sidx.at[pl.ds(0, 8)])
    pltpu.sync_copy(dst_hbm.at[pl.ds(gbase, 8)], didx.at[pl.ds(0, 8)])
    pltpu.async_copy(hs_hbm.at[sidx.at[0]], bufs.at[pl.ds(0, 128)], gsem)

    def body(j, _):
        u = (j % 2) * 128
        blk = j // 8
        r = j - blk * 8
        slot = (blk % 2) * 8

        pltpu.make_async_copy(hs_hbm.at[sidx.at[0]],
                              bufs.at[pl.ds(u, 128)], gsem).wait()

        @pl.when(j >= 1)
        def _():
            pltpu.make_async_copy(bufs.at[pl.ds(128 - u, 128)],
                                  acc.at[didx.at[0]], ssem).wait()

        @pl.when(jnp.logical_and(r == 0, blk + 1 < nblk))
        def _():
            pltpu.async_copy(src_hbm.at[pl.ds(gbase + (blk + 1) * 8, 8)],
                             sidx.at[pl.ds(8 - slot, 8)], isem)
            pltpu.async_copy(dst_hbm.at[pl.ds(gbase + (blk + 1) * 8, 8)],
                             didx.at[pl.ds(8 - slot, 8)], isem)

        @pl.when(jnp.logical_and(r == 7, j + 1 < gcnt))
        def _():
            pltpu.make_async_copy(src_hbm.at[pl.ds(gbase, 8)],
                                  sidx.at[pl.ds(0, 8)], isem).wait()
            pltpu.make_async_copy(dst_hbm.at[pl.ds(gbase, 8)],
                                  didx.at[pl.ds(0, 8)], isem).wait()

        @pl.when(j + 1 < gcnt)
        def _():
            blk1 = (j + 1) // 8
            srow = (blk1 % 2) * 8 + (j + 1 - blk1 * 8)
            pltpu.async_copy(hs_hbm.at[sidx.at[srow]],
                             bufs.at[pl.ds(128 - u, 128)], gsem)

        pltpu.async_copy(bufs.at[pl.ds(u, 128)],
                         acc.at[didx.at[slot + r]], ssem, add=True)
        return 0

    lax.fori_loop(0, gcnt, body, 0)
    pltpu.make_async_copy(bufs.at[pl.ds(128, 128)],
                          acc.at[didx.at[0]], ssem).wait()

    plsc.subcore_barrier()

    def wb_body(z, _):
        r0 = sid * ROWS_PT + z * 128
        pltpu.sync_copy(acc.at[pl.ds(r0, 128)], bufs.at[pl.ds(0, 128)])
        pltpu.sync_copy(bufs.at[pl.ds(0, 128)],
                        out_hbm.at[cid].at[pl.ds(r0, 128)])
        return 0

    lax.fori_loop(0, ROWS_PT // 128, wb_body, 0)


def _dis_dinv(degp):
    deg = jnp.sum(degp, axis=0) + 1.0
    dis = lax.rsqrt(deg)[:, None]
    dinv = (1.0 / deg)[:, None]
    return dis, dinv


def _mm1_body(x_ref, w_ref, degp_ref, h_ref, hs_ref):
    dis, _ = _dis_dinv(degp_ref[...])
    h = jnp.dot(x_ref[...], w_ref[...], preferred_element_type=jnp.float32)
    h_ref[...] = h
    hs_ref[...] = h * dis


def _mid_body(s_ref, h_ref, degp_ref, w_ref, b_ref, h2_ref, hs2_ref):
    dis, dinv = _dis_dinv(degp_ref[...])
    s = s_ref[0] + s_ref[1]
    a = jnp.maximum(s * dis + h_ref[...] * dinv + b_ref[...], 0.0)
    h2 = jnp.dot(a, w_ref[...], preferred_element_type=jnp.float32)
    h2_ref[...] = h2
    hs2_ref[...] = h2 * dis


def _out_body(s_ref, h_ref, degp_ref, b_ref, o_ref):
    dis, dinv = _dis_dinv(degp_ref[...])
    s = s_ref[0] + s_ref[1]
    o_ref[...] = jnp.maximum(s * dis + h_ref[...] * dinv + b_ref[...], 0.0)


def _row_spec(width):
    return pl.BlockSpec((MB, width), lambda i: (i, 0))


_degp_spec = pl.BlockSpec((NW, MB), lambda i: (0, i))
_s_spec = pl.BlockSpec((NC, MB, D_HID), lambda i: (0, i, 0))
_b_spec = pl.BlockSpec((1, D_HID), lambda i: (0, 0))


def _mm1(x, w1, degp):
    return pl.pallas_call(
        _mm1_body,
        grid=(GRID_M,),
        in_specs=[_row_spec(D_IN),
                  pl.BlockSpec((D_IN, D_HID), lambda i: (0, 0)),
                  _degp_spec],
        out_specs=[_row_spec(D_HID), _row_spec(D_HID)],
        out_shape=[jax.ShapeDtypeStruct((N_NODES, D_HID), jnp.float32),
                   jax.ShapeDtypeStruct((N_NODES, D_HID), jnp.float32)],
    )(x, w1, degp)


def _mid(s, h, degp, w2, b1):
    return pl.pallas_call(
        _mid_body,
        grid=(GRID_M,),
        in_specs=[_s_spec, _row_spec(D_HID), _degp_spec,
                  pl.BlockSpec((D_HID, D_HID), lambda i: (0, 0)), _b_spec],
        out_specs=[_row_spec(D_HID), _row_spec(D_HID)],
        out_shape=[jax.ShapeDtypeStruct((N_NODES, D_HID), jnp.float32),
                   jax.ShapeDtypeStruct((N_NODES, D_HID), jnp.float32)],
    )(s, h, degp, w2, b1)


def _out(s, h, degp, b2):
    return pl.pallas_call(
        _out_body,
        grid=(GRID_M,),
        in_specs=[_s_spec, _row_spec(D_HID), _degp_spec, _b_spec],
        out_specs=_row_spec(D_HID),
        out_shape=jax.ShapeDtypeStruct((N_NODES, D_HID), jnp.float32),
    )(s, h, degp, b2)


def kernel(x, edge_index, y, W1, b1, W2, b2):
    del y
    src = edge_index[0].astype(jnp.int32)
    dst = edge_index[1].astype(jnp.int32)
    pad_e = EPAD - N_EDGES
    src_p = jnp.concatenate([src, jnp.zeros((pad_e,), jnp.int32)])
    dst_p = jnp.concatenate([dst, jnp.full((pad_e,), NPAD - 1, jnp.int32)])
    src_m = src_p.reshape(EPAD // 128, 128)
    dst_m = dst_p.reshape(EPAD // 128, 128)

    b1r = b1.reshape(1, D_HID)
    b2r = b2.reshape(1, D_HID)

    degp = _deg_kernel(dst_p).reshape(NW, NPAD)
    h1, hs1 = _mm1(x, W1, degp)
    s1 = _scatter_kernel(hs1, src_m, dst_m)
    h2, hs2 = _mid(s1, h1, degp, W2, b1r)
    s2 = _scatter_kernel(hs2, src_m, dst_m)
    return _out(s2, h2, degp, b2r)


# speedup vs baseline: 1.1964x; 1.0031x over previous
import functools

import jax
import jax.numpy as jnp
from jax import lax
from jax.experimental import pallas as pl
from jax.experimental.pallas import tpu as pltpu
from jax.experimental.pallas import tpu_sc as plsc

N_NODES = 10000
N_EDGES = 320000
D_IN = 256
D_HID = 128

NC = 2
NS = 16
NW = NC * NS

NPAD = 10240
EPAD = 327680
EPT = EPAD // NW
GPT = EPT // 128
ROWS_PT = NPAD // NS

MB = 2048
GRID_M = NPAD // MB

_mesh = plsc.VectorSubcoreMesh(
    core_axis_name="c", subcore_axis_name="s", num_cores=NC, num_subcores=NS)


@functools.partial(
    pl.kernel,
    out_type=jax.ShapeDtypeStruct((NW, NPAD // 16, 16), jnp.float32),
    mesh=_mesh,
    compiler_params=pltpu.CompilerParams(needs_layout_passes=False),
    scratch_types=[
        pltpu.VMEM((NPAD // 16, 16), jnp.float32),
        pltpu.VMEM((2048,), jnp.int32),
    ],
)
def _deg_kernel(dst_hbm, out_hbm, acc, idxb):
    cid = lax.axis_index("c")
    sid = lax.axis_index("s")
    wid = cid * NS + sid

    z16 = jnp.zeros((16,), jnp.float32)

    def zero_body(i, _):
        acc[i, pl.ds(0, 16)] = z16
        return 0

    lax.fori_loop(0, NPAD // 16, zero_body, 0)

    ones16 = jnp.ones((16,), jnp.float32)

    def chunk_body(c, _):
        pltpu.sync_copy(dst_hbm.at[pl.ds(wid * EPT + c * 2048, 2048)], idxb)

        def grp_body(i, _):
            idx = idxb[pl.ds(i * 16, 16)]
            plsc.addupdate_scatter(
                acc, [lax.shift_right_logical(idx, 4),
                      jnp.bitwise_and(idx, 15)], ones16)
            return 0

        lax.fori_loop(0, 128, grp_body, 0)
        return 0

    lax.fori_loop(0, EPT // 2048, chunk_body, 0)
    pltpu.sync_copy(acc, out_hbm.at[wid])


G_SC0 = 152
G_SC1 = 2 * GPT - G_SC0


@functools.partial(
    pl.kernel,
    out_type=jax.ShapeDtypeStruct((NC, NPAD, D_HID), jnp.float32),
    mesh=_mesh,
    compiler_params=pltpu.CompilerParams(needs_layout_passes=False),
    scratch_types=[
        pltpu.VMEM((16, 128), jnp.int32),
        pltpu.VMEM((16, 128), jnp.int32),
        pltpu.VMEM((256, D_HID), jnp.float32),
        pltpu.VMEM_SHARED((NPAD, D_HID), jnp.float32),
        pltpu.SemaphoreType.DMA,
        pltpu.SemaphoreType.DMA,
        pltpu.SemaphoreType.DMA,
    ],
)
def _scatter_kernel(hs_hbm, src_hbm, dst_hbm, out_hbm,
                    sidx, didx, bufs, acc, gsem, ssem, isem):
    cid = lax.axis_index("c")
    sid = lax.axis_index("s")
    gcnt = jnp.where(cid == 0, G_SC0, G_SC1)
    gbase = jnp.where(cid == 0, sid * G_SC0, NS * G_SC0 + sid * G_SC1)
    nblk = gcnt // 8

    z16 = jnp.zeros((16,), jnp.float32)

    def zero_body(i, _):
        bufs[i >> 3, pl.ds((i & 7) * 16, 16)] = z16
        return 0

    lax.fori_loop(0, 128 * (D_HID // 16), zero_body, 0)

    def zcopy_body(z, _):
        pltpu.sync_copy(bufs.at[pl.ds(0, 128)],
                        acc.at[pl.ds(sid * ROWS_PT + z * 128, 128)])
        return 0

    lax.fori_loop(0, ROWS_PT // 128, zcopy_body, 0)
    plsc.subcore_barrier()

    pltpu.sync_copy(src_hbm.at[pl.ds(gbase, 8)], sidx.at[pl.ds(0, 8)])
    pltpu.sync_copy(dst_hbm.at[pl.ds(gbase, 8)], didx.at[pl.ds(0, 8)])
    pltpu.async_copy(hs_hbm.at[sidx.at[0]], bufs.at[pl.ds(0, 128)], gsem)

    def body(j, _):
        u = (j % 2) * 128
        blk = j // 8
        r = j - blk * 8
        slot = (blk % 2) * 8

        pltpu.make_async_copy(hs_hbm.at[sidx.at[0]],
                              bufs.at[pl.ds(u, 128)], gsem).wait()

        @pl.when(j >= 1)
        def _():
            pltpu.make_async_copy(bufs.at[pl.ds(128 - u, 128)],
                                  acc.at[didx.at[0]], ssem).wait()

        @pl.when(jnp.logical_and(r == 0, blk + 1 < nblk))
        def _():
            pltpu.async_copy(src_hbm.at[pl.ds(gbase + (blk + 1) * 8, 8)],
                             sidx.at[pl.ds(8 - slot, 8)], isem)
            pltpu.async_copy(dst_hbm.at[pl.ds(gbase + (blk + 1) * 8, 8)],
                             didx.at[pl.ds(8 - slot, 8)], isem)

        @pl.when(jnp.logical_and(r == 7, j + 1 < gcnt))
        def _():
            pltpu.make_async_copy(src_hbm.at[pl.ds(gbase, 8)],
                                  sidx.at[pl.ds(0, 8)], isem).wait()
            pltpu.make_async_copy(dst_hbm.at[pl.ds(gbase, 8)],
                                  didx.at[pl.ds(0, 8)], isem).wait()

        @pl.when(j + 1 < gcnt)
        def _():
            blk1 = (j + 1) // 8
            srow = (blk1 % 2) * 8 + (j + 1 - blk1 * 8)
            pltpu.async_copy(hs_hbm.at[sidx.at[srow]],
                             bufs.at[pl.ds(128 - u, 128)], gsem)

        pltpu.async_copy(bufs.at[pl.ds(u, 128)],
                         acc.at[didx.at[slot + r]], ssem, add=True)
        return 0

    lax.fori_loop(0, gcnt, body, 0)
    pltpu.make_async_copy(bufs.at[pl.ds(128, 128)],
                          acc.at[didx.at[0]], ssem).wait()

    plsc.subcore_barrier()

    def wb_body(z, _):
        r0 = sid * ROWS_PT + z * 128
        pltpu.sync_copy(acc.at[pl.ds(r0, 128)], bufs.at[pl.ds(0, 128)])
        pltpu.sync_copy(bufs.at[pl.ds(0, 128)],
                        out_hbm.at[cid].at[pl.ds(r0, 128)])
        return 0

    lax.fori_loop(0, ROWS_PT // 128, wb_body, 0)


def _dis_dinv(degp):
    deg = jnp.sum(degp, axis=0) + 1.0
    dis = lax.rsqrt(deg)[:, None]
    dinv = (1.0 / deg)[:, None]
    return dis, dinv


def _mm1_body(x_ref, w_ref, degp_ref, h_ref, hs_ref):
    dis, _ = _dis_dinv(degp_ref[...])
    h = jnp.dot(x_ref[...], w_ref[...], preferred_element_type=jnp.float32)
    h_ref[...] = h
    hs_ref[...] = h * dis


def _mid_body(s_ref, h_ref, degp_ref, w_ref, b_ref, h2_ref, hs2_ref):
    dis, dinv = _dis_dinv(degp_ref[...])
    s = s_ref[0] + s_ref[1]
    a = jnp.maximum(s * dis + h_ref[...] * dinv + b_ref[...], 0.0)
    h2 = jnp.dot(a, w_ref[...], preferred_element_type=jnp.float32)
    h2_ref[...] = h2
    hs2_ref[...] = h2 * dis


def _out_body(s_ref, h_ref, degp_ref, b_ref, o_ref):
    dis, dinv = _dis_dinv(degp_ref[...])
    s = s_ref[0] + s_ref[1]
    o_ref[...] = jnp.maximum(s * dis + h_ref[...] * dinv + b_ref[...], 0.0)


def _row_spec(width):
    return pl.BlockSpec((MB, width), lambda i: (i, 0))


_degp_spec = pl.BlockSpec((NW, MB), lambda i: (0, i))
_s_spec = pl.BlockSpec((NC, MB, D_HID), lambda i: (0, i, 0))
_b_spec = pl.BlockSpec((1, D_HID), lambda i: (0, 0))


def _mm1(x, w1, degp):
    return pl.pallas_call(
        _mm1_body,
        grid=(GRID_M,),
        in_specs=[_row_spec(D_IN),
                  pl.BlockSpec((D_IN, D_HID), lambda i: (0, 0)),
                  _degp_spec],
        out_specs=[_row_spec(D_HID), _row_spec(D_HID)],
        out_shape=[jax.ShapeDtypeStruct((N_NODES, D_HID), jnp.float32),
                   jax.ShapeDtypeStruct((N_NODES, D_HID), jnp.float32)],
    )(x, w1, degp)


def _mid(s, h, degp, w2, b1):
    return pl.pallas_call(
        _mid_body,
        grid=(GRID_M,),
        in_specs=[_s_spec, _row_spec(D_HID), _degp_spec,
                  pl.BlockSpec((D_HID, D_HID), lambda i: (0, 0)), _b_spec],
        out_specs=[_row_spec(D_HID), _row_spec(D_HID)],
        out_shape=[jax.ShapeDtypeStruct((N_NODES, D_HID), jnp.float32),
                   jax.ShapeDtypeStruct((N_NODES, D_HID), jnp.float32)],
    )(s, h, degp, w2, b1)


def _out(s, h, degp, b2):
    return pl.pallas_call(
        _out_body,
        grid=(GRID_M,),
        in_specs=[_s_spec, _row_spec(D_HID), _degp_spec, _b_spec],
        out_specs=_row_spec(D_HID),
        out_shape=jax.ShapeDtypeStruct((N_NODES, D_HID), jnp.float32),
    )(s, h, degp, b2)


def kernel(x, edge_index, y, W1, b1, W2, b2):
    del y
    src = edge_index[0].astype(jnp.int32)
    dst = edge_index[1].astype(jnp.int32)
    pad_e = EPAD - N_EDGES
    src_p = jnp.concatenate([src, jnp.zeros((pad_e,), jnp.int32)])
    dst_p = jnp.concatenate([dst, jnp.full((pad_e,), NPAD - 1, jnp.int32)])
    src_m = src_p.reshape(EPAD // 128, 128)
    dst_m = dst_p.reshape(EPAD // 128, 128)

    b1r = b1.reshape(1, D_HID)
    b2r = b2.reshape(1, D_HID)

    degp = _deg_kernel(dst_p).reshape(NW, NPAD)
    h1, hs1 = _mm1(x, W1, degp)
    s1 = _scatter_kernel(hs1, src_m, dst_m)
    h2, hs2 = _mid(s1, h1, degp, W2, b1r)
    s2 = _scatter_kernel(hs2, src_m, dst_m)
    return _out(s2, h2, degp, b2r)
